# baseline (speedup 1.0000x reference)
import jax
import jax.numpy as jnp
from jax.experimental import pallas as pl
from jax.experimental.pallas import tpu as pltpu

_BN = 512
_AP = 128


def _fused(s_ref, w1_ref, b1_ref, w2_ref, b2_ref, probs_ref, done_ref):
    s = s_ref[...]
    d = s.shape[1]
    side = 32 if d == 1024 else int(round(d ** 0.5))

    h = jnp.dot(s, w1_ref[...], preferred_element_type=jnp.float32)
    h = jnp.maximum(h + b1_ref[...][None, :], 0.0)
    logits = jnp.dot(h, w2_ref[...], preferred_element_type=jnp.float32)
    logits = logits + b2_ref[...][None, :]

    lane = jax.lax.broadcasted_iota(jnp.int32, logits.shape, 1)
    logits = jnp.where(lane < 3, logits, jnp.float32(-1e30))
    m = jnp.max(logits, axis=1, keepdims=True)
    e = jnp.exp(logits - m)
    p = e / jnp.sum(e, axis=1, keepdims=True)

    mx = jnp.max(s, axis=1, keepdims=True)
    col = jax.lax.broadcasted_iota(jnp.int32, s.shape, 1)
    idx = jnp.min(jnp.where(s == mx, col, d), axis=1)
    x = idx % side
    y = idx // side
    md = (y < side - 1).astype(jnp.float32)[:, None]
    mr = (x < side - 1).astype(jnp.float32)[:, None]
    mask = jnp.where(lane == 0, md,
                     jnp.where(lane == 1, mr,
                               jnp.where(lane == 2, 1.0, 0.0)))

    p = mask * (p + 1e-8)
    p = p / jnp.sum(p, axis=1, keepdims=True)
    probs_ref[...] = p
    done_ref[...] = (idx == d - 1).astype(jnp.float32)


def kernel(s, W1, b1, W2, b2):
    n, d = s.shape
    hdim = W1.shape[1]
    a = W2.shape[1]
    w2p = jnp.pad(W2, ((0, 0), (0, _AP - a)))
    b2p = jnp.pad(b2, (0, _AP - a))

    probs_p, done_f = pl.pallas_call(
        _fused,
        grid=(n // _BN,),
        in_specs=[
            pl.BlockSpec((_BN, d), lambda i: (i, 0)),
            pl.BlockSpec((d, hdim), lambda i: (0, 0)),
            pl.BlockSpec((hdim,), lambda i: (0,)),
            pl.BlockSpec((hdim, _AP), lambda i: (0, 0)),
            pl.BlockSpec((_AP,), lambda i: (0,)),
        ],
        out_specs=[
            pl.BlockSpec((_BN, _AP), lambda i: (i, 0)),
            pl.BlockSpec((_BN,), lambda i: (i,)),
        ],
        out_shape=[
            jax.ShapeDtypeStruct((n, _AP), jnp.float32),
            jax.ShapeDtypeStruct((n,), jnp.float32),
        ],
        compiler_params=pltpu.CompilerParams(
            dimension_semantics=("parallel",),
        ),
    )(s, W1, b1, w2p, b2p)

    return probs_p[:, :a], done_f > 0.5


# speedup vs baseline: 1.0062x; 1.0062x over previous
import jax
import jax.numpy as jnp
from jax.experimental import pallas as pl
from jax.experimental.pallas import tpu as pltpu

_BN = 512
_AP = 128


def _fused(s_ref, w1_ref, b1_ref, w2_ref, b2_ref, probs_ref, done_ref):
    s = s_ref[...]
    d = s.shape[1]
    side = 32 if d == 1024 else int(round(d ** 0.5))

    h = jnp.dot(s.astype(jnp.bfloat16), w1_ref[...].astype(jnp.bfloat16),
                preferred_element_type=jnp.float32)
    h = jnp.maximum(h + b1_ref[...][None, :], 0.0)
    logits = jnp.dot(h.astype(jnp.bfloat16), w2_ref[...].astype(jnp.bfloat16),
                     preferred_element_type=jnp.float32)
    logits = logits + b2_ref[...][None, :]

    lane = jax.lax.broadcasted_iota(jnp.int32, logits.shape, 1)
    logits = jnp.where(lane < 3, logits, jnp.float32(-1e30))
    m = jnp.max(logits, axis=1, keepdims=True)
    e = jnp.exp(logits - m)
    p = e / jnp.sum(e, axis=1, keepdims=True)

    mx = jnp.max(s, axis=1, keepdims=True)
    col = jax.lax.broadcasted_iota(jnp.int32, s.shape, 1)
    idx = jnp.min(jnp.where(s == mx, col, d), axis=1)
    x = idx % side
    y = idx // side
    md = (y < side - 1).astype(jnp.float32)[:, None]
    mr = (x < side - 1).astype(jnp.float32)[:, None]
    mask = jnp.where(lane == 0, md,
                     jnp.where(lane == 1, mr,
                               jnp.where(lane == 2, 1.0, 0.0)))

    p = mask * (p + 1e-8)
    p = p / jnp.sum(p, axis=1, keepdims=True)
    probs_ref[...] = p
    done_ref[...] = (idx == d - 1).astype(jnp.float32)


def kernel(s, W1, b1, W2, b2):
    n, d = s.shape
    hdim = W1.shape[1]
    a = W2.shape[1]
    w2p = jnp.pad(W2, ((0, 0), (0, _AP - a)))
    b2p = jnp.pad(b2, (0, _AP - a))

    probs_p, done_f = pl.pallas_call(
        _fused,
        grid=(n // _BN,),
        in_specs=[
            pl.BlockSpec((_BN, d), lambda i: (i, 0)),
            pl.BlockSpec((d, hdim), lambda i: (0, 0)),
            pl.BlockSpec((hdim,), lambda i: (0,)),
            pl.BlockSpec((hdim, _AP), lambda i: (0, 0)),
            pl.BlockSpec((_AP,), lambda i: (0,)),
        ],
        out_specs=[
            pl.BlockSpec((_BN, _AP), lambda i: (i, 0)),
            pl.BlockSpec((_BN,), lambda i: (i,)),
        ],
        out_shape=[
            jax.ShapeDtypeStruct((n, _AP), jnp.float32),
            jax.ShapeDtypeStruct((n,), jnp.float32),
        ],
        compiler_params=pltpu.CompilerParams(
            dimension_semantics=("parallel",),
        ),
    )(s, W1, b1, w2p, b2p)

    return probs_p[:, :a], done_f > 0.5
